# Initial kernel scaffold; baseline (speedup 1.0000x reference)
import jax
import jax.numpy as jnp
from jax.experimental import pallas as pl


def kernel(q_in, k_in, v_in, Wq, bq):
    raise NotImplementedError("write your pallas kernel here")


# speedup vs baseline: 37.7058x; 37.7058x over previous
import functools

import jax
import jax.numpy as jnp
import numpy as np
from jax import lax
from jax.experimental import pallas as pl
from jax.experimental.pallas import tpu as pltpu
from jax.experimental.pallas import tpu_sc as plsc

B, S, D = 4, 2048, 1024
DK = 64
H = 16
KK = 15
NR = B * DK
DCH = 4
FB = 256
NFB = S // FB
PREC = lax.Precision.HIGHEST
_TWO_PI_OVER_S = float(2.0 * np.pi / S)

SC_NC = 2
SC_NS = 16
SC_NW = SC_NC * SC_NS
ROWS_PER_W = NR // SC_NW


def _proj_kernel(q_ref, k_ref, v_ref, wq_ref, bqr_ref, bqc_ref,
                 qa_ref, ka_ref, vt_ref):
    d = pl.program_id(1)
    w = wq_ref[...].astype(jnp.bfloat16)
    qp = jnp.dot(q_ref[0].astype(jnp.bfloat16), w,
                 preferred_element_type=jnp.float32)
    kp = jnp.dot(k_ref[0].astype(jnp.bfloat16), w,
                 preferred_element_type=jnp.float32)
    vp = lax.dot_general(w, v_ref[0].astype(jnp.bfloat16),
                         (((0,), (1,)), ((), ())),
                         preferred_element_type=jnp.float32)

    @pl.when(d == 0)
    def _():
        qa_ref[0] = qp + bqr_ref[...]
        ka_ref[0] = kp + bqr_ref[...]
        vt_ref[0] = vp + bqc_ref[...]

    @pl.when(d != 0)
    def _():
        qa_ref[0] = qa_ref[0] + qp
        ka_ref[0] = ka_ref[0] + kp
        vt_ref[0] = vt_ref[0] + vp


def _dft_block(row0):
    r_idx = lax.broadcasted_iota(jnp.int32, (FB, S), 0) + row0
    t_idx = lax.broadcasted_iota(jnp.int32, (FB, S), 1)
    prod = (r_idx * t_idx) & (S - 1)
    theta = prod.astype(jnp.float32) * _TWO_PI_OVER_S
    return jnp.cos(theta), jnp.sin(theta)


def _fwd1_kernel(qa_ref, ka_ref, sr_ref, si_ref):
    fc, fs = _dft_block(pl.program_id(0) * FB)
    qa = jnp.concatenate([qa_ref[b] for b in range(B)], axis=1)
    ka = jnp.concatenate([ka_ref[b] for b in range(B)], axis=1)
    qr = jnp.dot(fc, qa, preferred_element_type=jnp.float32, precision=PREC)
    qi = -jnp.dot(fs, qa, preferred_element_type=jnp.float32, precision=PREC)
    kr = jnp.dot(fc, ka, preferred_element_type=jnp.float32, precision=PREC)
    ki = -jnp.dot(fs, ka, preferred_element_type=jnp.float32, precision=PREC)
    sr_ref[...] = qr * kr + qi * ki
    si_ref[...] = qi * kr - qr * ki


def _inv1_kernel(sr_ref, si_ref, abs_ref):
    fc, fs = _dft_block(pl.program_id(0) * FB)
    c = jnp.dot(fc, sr_ref[...], preferred_element_type=jnp.float32,
                precision=PREC)
    d = jnp.dot(fs, si_ref[...], preferred_element_type=jnp.float32,
                precision=PREC)
    abs_ref[...] = jnp.abs((c - d) * (1.0 / S))


def _topk_kernel(abs_ref, w_ref, lag_ref):
    x = abs_ref[...]
    iota = lax.broadcasted_iota(jnp.int32, (S, NR), 0)
    vals = []
    lags = []
    for _ in range(KK):
        m = jnp.max(x, axis=0, keepdims=True)
        hit = x == m
        am = jnp.min(jnp.where(hit, iota, S), axis=0, keepdims=True)
        vals.append(m)
        lags.append(am)
        x = jnp.where(iota == am, -1.0, x)
    v15 = jnp.concatenate(vals, axis=0)
    e = jnp.exp(v15 - v15[0:1])
    w15 = e / jnp.sum(e, axis=0, keepdims=True)
    w_ref[...] = jnp.concatenate(
        [w15, jnp.zeros((1, NR), jnp.float32)], axis=0)
    lag_ref[...] = jnp.concatenate(
        lags + [jnp.zeros((1, NR), jnp.int32)], axis=0)


@functools.cache
def _make_agg_sc():
    mesh = plsc.VectorSubcoreMesh(core_axis_name="c", subcore_axis_name="s",
                                  num_cores=SC_NC)

    @functools.partial(
        pl.kernel,
        out_type=jax.ShapeDtypeStruct((NR, S), jnp.float32),
        mesh=mesh,
        compiler_params=pltpu.CompilerParams(needs_layout_passes=False),
        scratch_types=[
            pltpu.VMEM((S,), jnp.float32),
            pltpu.VMEM((S,), jnp.float32),
            pltpu.VMEM((16 * 16,), jnp.int32),
            pltpu.VMEM((16 * 16,), jnp.float32),
        ],
    )
    def agg(vt_hbm, lag_hbm, w_hbm, out_hbm, vrow, orow, lrow, wrow):
        wid = lax.axis_index("s") * SC_NC + lax.axis_index("c")
        base = wid * ROWS_PER_W
        lane = lax.iota(jnp.int32, 16)

        def row_body(j, carry):
            r = base + j
            pltpu.sync_copy(vt_hbm.at[r], vrow)
            pltpu.sync_copy(lag_hbm.at[r], lrow)
            pltpu.sync_copy(w_hbm.at[r], wrow)
            lag_b = [lrow[pl.ds(16 * i, 16)] for i in range(16)]
            w_b = [wrow[pl.ds(16 * i, 16)] for i in range(16)]

            def chunk_body(jc, carry2):
                basei = lane + jc * 16
                acc = jnp.zeros((16,), jnp.float32)
                for i in range(16):
                    idx = (basei + lag_b[i]) & (S - 1)
                    acc = acc + plsc.load_gather(vrow, [idx]) * w_b[i]
                orow[pl.ds(jc * 16, 16)] = acc
                return carry2

            lax.fori_loop(0, S // 16, chunk_body, 0)
            pltpu.sync_copy(orow, out_hbm.at[r])
            return carry

        lax.fori_loop(0, ROWS_PER_W, row_body, 0)

    return agg


def _tile_kernel(agg_ref, out_ref):
    a = agg_ref[0]
    eye = jnp.eye(DK, dtype=jnp.float32)
    at = lax.dot_general(a, eye, (((0,), (0,)), ((), ())),
                         preferred_element_type=jnp.float32,
                         precision=PREC)
    out_ref[0] = jnp.concatenate([at] * H, axis=1)


def kernel(q_in, k_in, v_in, Wq, bq):
    dch = D // DCH
    qa, ka, vt = pl.pallas_call(
        _proj_kernel,
        grid=(B, DCH),
        in_specs=[
            pl.BlockSpec((1, S, dch), lambda b, d: (b, 0, d)),
            pl.BlockSpec((1, S, dch), lambda b, d: (b, 0, d)),
            pl.BlockSpec((1, S, dch), lambda b, d: (b, 0, d)),
            pl.BlockSpec((dch, DK), lambda b, d: (d, 0)),
            pl.BlockSpec((1, DK), lambda b, d: (0, 0)),
            pl.BlockSpec((DK, 1), lambda b, d: (0, 0)),
        ],
        out_specs=[
            pl.BlockSpec((1, S, DK), lambda b, d: (b, 0, 0)),
            pl.BlockSpec((1, S, DK), lambda b, d: (b, 0, 0)),
            pl.BlockSpec((1, DK, S), lambda b, d: (b, 0, 0)),
        ],
        out_shape=[
            jax.ShapeDtypeStruct((B, S, DK), jnp.float32),
            jax.ShapeDtypeStruct((B, S, DK), jnp.float32),
            jax.ShapeDtypeStruct((B, DK, S), jnp.float32),
        ],
    )(q_in, k_in, v_in, Wq, bq.reshape(1, DK), bq.reshape(DK, 1))

    sr, si = pl.pallas_call(
        _fwd1_kernel,
        grid=(NFB,),
        in_specs=[
            pl.BlockSpec((B, S, DK), lambda f: (0, 0, 0)),
            pl.BlockSpec((B, S, DK), lambda f: (0, 0, 0)),
        ],
        out_specs=[
            pl.BlockSpec((FB, NR), lambda f: (f, 0)),
            pl.BlockSpec((FB, NR), lambda f: (f, 0)),
        ],
        out_shape=[
            jax.ShapeDtypeStruct((S, NR), jnp.float32),
            jax.ShapeDtypeStruct((S, NR), jnp.float32),
        ],
    )(qa, ka)

    qk_abs = pl.pallas_call(
        _inv1_kernel,
        grid=(NFB,),
        in_specs=[
            pl.BlockSpec((S, NR), lambda t: (0, 0)),
            pl.BlockSpec((S, NR), lambda t: (0, 0)),
        ],
        out_specs=pl.BlockSpec((FB, NR), lambda t: (t, 0)),
        out_shape=jax.ShapeDtypeStruct((S, NR), jnp.float32),
    )(sr, si)

    w16, lag16 = pl.pallas_call(
        _topk_kernel,
        out_shape=[
            jax.ShapeDtypeStruct((16, NR), jnp.float32),
            jax.ShapeDtypeStruct((16, NR), jnp.int32),
        ],
    )(qk_abs)

    lag_bc = jnp.broadcast_to(lag16.T[:, :, None], (NR, 16, 16)).reshape(NR, 256)
    w_bc = jnp.broadcast_to(w16.T[:, :, None], (NR, 16, 16)).reshape(NR, 256)
    agg = _make_agg_sc()(vt.reshape(NR, S), lag_bc, w_bc)

    out = pl.pallas_call(
        _tile_kernel,
        grid=(B,),
        in_specs=[pl.BlockSpec((1, DK, S), lambda b: (b, 0, 0))],
        out_specs=pl.BlockSpec((1, S, H * DK), lambda b: (b, 0, 0)),
        out_shape=jax.ShapeDtypeStruct((B, S, H * DK), jnp.float32),
    )(agg.reshape(B, DK, S))
    return out


# speedup vs baseline: 54.2660x; 1.4392x over previous
import functools

import jax
import jax.numpy as jnp
import numpy as np
from jax import lax
from jax.experimental import pallas as pl
from jax.experimental.pallas import tpu as pltpu
from jax.experimental.pallas import tpu_sc as plsc

B, S, D = 4, 2048, 1024
DK = 64
H = 16
KK = 15
NR = B * DK
DCH = 4
FB = 256
NFB = S // FB
PREC = lax.Precision.HIGHEST
NF = S // 2
NFB2 = NF // FB
_TWO_PI_OVER_S = float(2.0 * np.pi / S)

SC_NC = 2
SC_NS = 16
SC_NW = SC_NC * SC_NS
ROWS_PER_W = NR // SC_NW


def _proj_kernel(q_ref, k_ref, v_ref, wq_ref, bqr_ref, bqc_ref,
                 qa_ref, ka_ref, vt_ref):
    d = pl.program_id(1)
    w = wq_ref[...].astype(jnp.bfloat16)
    qp = jnp.dot(q_ref[0].astype(jnp.bfloat16), w,
                 preferred_element_type=jnp.float32)
    kp = jnp.dot(k_ref[0].astype(jnp.bfloat16), w,
                 preferred_element_type=jnp.float32)
    vp = lax.dot_general(w, v_ref[0].astype(jnp.bfloat16),
                         (((0,), (1,)), ((), ())),
                         preferred_element_type=jnp.float32)

    @pl.when(d == 0)
    def _():
        qa_ref[0] = qp + bqr_ref[...]
        ka_ref[0] = kp + bqr_ref[...]
        vt_ref[0] = vp + bqc_ref[...]

    @pl.when(d != 0)
    def _():
        qa_ref[0] = qa_ref[0] + qp
        ka_ref[0] = ka_ref[0] + kp
        vt_ref[0] = vt_ref[0] + vp


def _dft_block(row0):
    r_idx = lax.broadcasted_iota(jnp.int32, (FB, S), 0) + row0
    t_idx = lax.broadcasted_iota(jnp.int32, (FB, S), 1)
    prod = (r_idx * t_idx) & (S - 1)
    theta = prod.astype(jnp.float32) * _TWO_PI_OVER_S
    return jnp.cos(theta), jnp.sin(theta)


def _dft_block_rect(row0):
    r_idx = lax.broadcasted_iota(jnp.int32, (FB, NF), 0) + row0
    f_idx = lax.broadcasted_iota(jnp.int32, (FB, NF), 1)
    prod = (r_idx * f_idx) & (S - 1)
    theta = prod.astype(jnp.float32) * _TWO_PI_OVER_S
    return jnp.cos(theta), jnp.sin(theta)


def _fwd1_kernel(qa_ref, ka_ref, sr_ref, si_ref, nyq_ref):
    fb = pl.program_id(0)
    fc, fs = _dft_block(fb * FB)
    qa = jnp.concatenate([qa_ref[b] for b in range(B)], axis=1)
    ka = jnp.concatenate([ka_ref[b] for b in range(B)], axis=1)
    qr = jnp.dot(fc, qa, preferred_element_type=jnp.float32, precision=PREC)
    qi = -jnp.dot(fs, qa, preferred_element_type=jnp.float32, precision=PREC)
    kr = jnp.dot(fc, ka, preferred_element_type=jnp.float32, precision=PREC)
    ki = -jnp.dot(fs, ka, preferred_element_type=jnp.float32, precision=PREC)
    f_col = lax.broadcasted_iota(jnp.int32, (FB, 1), 0) + fb * FB
    sc = jnp.where(f_col == 0, 1.0 / S, 2.0 / S)
    sr_ref[...] = (qr * kr + qi * ki) * sc
    si_ref[...] = (qi * kr - qr * ki) * sc

    @pl.when(fb == 0)
    def _():
        alt = (1 - 2 * (lax.broadcasted_iota(jnp.int32, (S, 1), 0) & 1)
               ).astype(jnp.float32)
        qn = jnp.sum(qa * alt, axis=0, keepdims=True)
        kn = jnp.sum(ka * alt, axis=0, keepdims=True)
        nyq_ref[...] = qn * kn * (1.0 / S)


def _inv1_kernel(sr_ref, si_ref, nyq_ref, abs_ref):
    tb = pl.program_id(0)
    fc, fs = _dft_block_rect(tb * FB)
    c = jnp.dot(fc, sr_ref[...], preferred_element_type=jnp.float32,
                precision=PREC)
    d = jnp.dot(fs, si_ref[...], preferred_element_type=jnp.float32,
                precision=PREC)
    alt = (1 - 2 * (lax.broadcasted_iota(jnp.int32, (FB, 1), 0) + tb * FB & 1)
           ).astype(jnp.float32)
    abs_ref[...] = jnp.abs(c - d + alt * nyq_ref[...])


def _topk_kernel(abs_ref, w_ref, lag_ref):
    x = abs_ref[...]
    iota = lax.broadcasted_iota(jnp.int32, (S, NR), 0)
    vals = []
    lags = []
    for _ in range(KK):
        m = jnp.max(x, axis=0, keepdims=True)
        hit = x == m
        am = jnp.min(jnp.where(hit, iota, S), axis=0, keepdims=True)
        vals.append(m)
        lags.append(am)
        x = jnp.where(iota == am, -1.0, x)
    v15 = jnp.concatenate(vals, axis=0)
    e = jnp.exp(v15 - v15[0:1])
    w15 = e / jnp.sum(e, axis=0, keepdims=True)
    w_ref[...] = jnp.concatenate(
        [w15, jnp.zeros((1, NR), jnp.float32)], axis=0)
    lag_ref[...] = jnp.concatenate(
        lags + [jnp.zeros((1, NR), jnp.int32)], axis=0)


@functools.cache
def _make_agg_sc():
    mesh = plsc.VectorSubcoreMesh(core_axis_name="c", subcore_axis_name="s",
                                  num_cores=SC_NC)

    @functools.partial(
        pl.kernel,
        out_type=jax.ShapeDtypeStruct((NR, S), jnp.float32),
        mesh=mesh,
        compiler_params=pltpu.CompilerParams(needs_layout_passes=False),
        scratch_types=[
            pltpu.VMEM((S,), jnp.float32),
            pltpu.VMEM((S,), jnp.float32),
            pltpu.VMEM((16 * 16,), jnp.int32),
            pltpu.VMEM((16 * 16,), jnp.float32),
        ],
    )
    def agg(vt_hbm, lag_hbm, w_hbm, out_hbm, vrow, orow, lrow, wrow):
        wid = lax.axis_index("s") * SC_NC + lax.axis_index("c")
        base = wid * ROWS_PER_W
        lane = lax.iota(jnp.int32, 16)

        def row_body(j, carry):
            r = base + j
            pltpu.sync_copy(vt_hbm.at[r], vrow)
            pltpu.sync_copy(lag_hbm.at[r], lrow)
            pltpu.sync_copy(w_hbm.at[r], wrow)
            lag_b = [lrow[pl.ds(16 * i, 16)] for i in range(16)]
            w_b = [wrow[pl.ds(16 * i, 16)] for i in range(16)]

            def chunk_body(jc, carry2):
                basei = lane + jc * 16
                acc = jnp.zeros((16,), jnp.float32)
                for i in range(16):
                    idx = (basei + lag_b[i]) & (S - 1)
                    acc = acc + plsc.load_gather(vrow, [idx]) * w_b[i]
                orow[pl.ds(jc * 16, 16)] = acc
                return carry2

            lax.fori_loop(0, S // 16, chunk_body, 0)
            pltpu.sync_copy(orow, out_hbm.at[r])
            return carry

        lax.fori_loop(0, ROWS_PER_W, row_body, 0)

    return agg


def _tile_kernel(agg_ref, out_ref):
    a = agg_ref[0]
    eye = jnp.eye(DK, dtype=jnp.float32)
    at = lax.dot_general(a, eye, (((0,), (0,)), ((), ())),
                         preferred_element_type=jnp.float32,
                         precision=PREC)
    out_ref[0] = jnp.concatenate([at] * H, axis=1)


def kernel(q_in, k_in, v_in, Wq, bq):
    dch = D // DCH
    qa, ka, vt = pl.pallas_call(
        _proj_kernel,
        grid=(B, DCH),
        in_specs=[
            pl.BlockSpec((1, S, dch), lambda b, d: (b, 0, d)),
            pl.BlockSpec((1, S, dch), lambda b, d: (b, 0, d)),
            pl.BlockSpec((1, S, dch), lambda b, d: (b, 0, d)),
            pl.BlockSpec((dch, DK), lambda b, d: (d, 0)),
            pl.BlockSpec((1, DK), lambda b, d: (0, 0)),
            pl.BlockSpec((DK, 1), lambda b, d: (0, 0)),
        ],
        out_specs=[
            pl.BlockSpec((1, S, DK), lambda b, d: (b, 0, 0)),
            pl.BlockSpec((1, S, DK), lambda b, d: (b, 0, 0)),
            pl.BlockSpec((1, DK, S), lambda b, d: (b, 0, 0)),
        ],
        out_shape=[
            jax.ShapeDtypeStruct((B, S, DK), jnp.float32),
            jax.ShapeDtypeStruct((B, S, DK), jnp.float32),
            jax.ShapeDtypeStruct((B, DK, S), jnp.float32),
        ],
    )(q_in, k_in, v_in, Wq, bq.reshape(1, DK), bq.reshape(DK, 1))

    sr, si, nyq = pl.pallas_call(
        _fwd1_kernel,
        grid=(NFB2,),
        in_specs=[
            pl.BlockSpec((B, S, DK), lambda f: (0, 0, 0)),
            pl.BlockSpec((B, S, DK), lambda f: (0, 0, 0)),
        ],
        out_specs=[
            pl.BlockSpec((FB, NR), lambda f: (f, 0)),
            pl.BlockSpec((FB, NR), lambda f: (f, 0)),
            pl.BlockSpec((1, NR), lambda f: (0, 0)),
        ],
        out_shape=[
            jax.ShapeDtypeStruct((NF, NR), jnp.float32),
            jax.ShapeDtypeStruct((NF, NR), jnp.float32),
            jax.ShapeDtypeStruct((1, NR), jnp.float32),
        ],
    )(qa, ka)

    qk_abs = pl.pallas_call(
        _inv1_kernel,
        grid=(NFB,),
        in_specs=[
            pl.BlockSpec((NF, NR), lambda t: (0, 0)),
            pl.BlockSpec((NF, NR), lambda t: (0, 0)),
            pl.BlockSpec((1, NR), lambda t: (0, 0)),
        ],
        out_specs=pl.BlockSpec((FB, NR), lambda t: (t, 0)),
        out_shape=jax.ShapeDtypeStruct((S, NR), jnp.float32),
    )(sr, si, nyq)

    w16, lag16 = pl.pallas_call(
        _topk_kernel,
        out_shape=[
            jax.ShapeDtypeStruct((16, NR), jnp.float32),
            jax.ShapeDtypeStruct((16, NR), jnp.int32),
        ],
    )(qk_abs)

    lag_bc = jnp.broadcast_to(lag16.T[:, :, None], (NR, 16, 16)).reshape(NR, 256)
    w_bc = jnp.broadcast_to(w16.T[:, :, None], (NR, 16, 16)).reshape(NR, 256)
    agg = _make_agg_sc()(vt.reshape(NR, S), lag_bc, w_bc)

    out = pl.pallas_call(
        _tile_kernel,
        grid=(B,),
        in_specs=[pl.BlockSpec((1, DK, S), lambda b: (b, 0, 0))],
        out_specs=pl.BlockSpec((1, S, H * DK), lambda b: (b, 0, 0)),
        out_shape=jax.ShapeDtypeStruct((B, S, H * DK), jnp.float32),
    )(agg.reshape(B, DK, S))
    return out


# speedup vs baseline: 60.5578x; 1.1159x over previous
import functools

import jax
import jax.numpy as jnp
import numpy as np
from jax import lax
from jax.experimental import pallas as pl
from jax.experimental.pallas import tpu as pltpu
from jax.experimental.pallas import tpu_sc as plsc

B, S, D = 4, 2048, 1024
DK = 64
H = 16
KK = 15
NR = B * DK
DCH = 4
FB = 256
NFB = S // FB
PREC = lax.Precision.HIGHEST
NF = S // 2
NFB2 = NF // FB
_TWO_PI_OVER_S = float(2.0 * np.pi / S)

SC_NC = 2
SC_NS = 16
SC_NW = SC_NC * SC_NS
ROWS_PER_W = NR // SC_NW


def _proj_kernel(q_ref, k_ref, v_ref, wq_ref, bqr_ref, bqc_ref,
                 qa_ref, ka_ref, vt_ref):
    d = pl.program_id(1)
    w = wq_ref[...].astype(jnp.bfloat16)
    qp = jnp.dot(q_ref[0].astype(jnp.bfloat16), w,
                 preferred_element_type=jnp.float32)
    kp = jnp.dot(k_ref[0].astype(jnp.bfloat16), w,
                 preferred_element_type=jnp.float32)
    vp = lax.dot_general(w, v_ref[0].astype(jnp.bfloat16),
                         (((0,), (1,)), ((), ())),
                         preferred_element_type=jnp.float32)

    @pl.when(d == 0)
    def _():
        qa_ref[0] = qp + bqr_ref[...]
        ka_ref[0] = kp + bqr_ref[...]
        vt_ref[0] = vp + bqc_ref[...]

    @pl.when(d != 0)
    def _():
        qa_ref[0] = qa_ref[0] + qp
        ka_ref[0] = ka_ref[0] + kp
        vt_ref[0] = vt_ref[0] + vp


def _dft_block(row0):
    r_idx = lax.broadcasted_iota(jnp.int32, (FB, S), 0) + row0
    t_idx = lax.broadcasted_iota(jnp.int32, (FB, S), 1)
    prod = (r_idx * t_idx) & (S - 1)
    theta = prod.astype(jnp.float32) * _TWO_PI_OVER_S
    return jnp.cos(theta), jnp.sin(theta)


def _dot3(a, b):
    a_hi = a.astype(jnp.bfloat16)
    a_lo = (a - a_hi.astype(jnp.float32)).astype(jnp.bfloat16)
    b_hi = b.astype(jnp.bfloat16)
    b_lo = (b - b_hi.astype(jnp.float32)).astype(jnp.bfloat16)

    def d(x, y):
        return jnp.dot(x, y, preferred_element_type=jnp.float32)

    return d(a_hi, b_hi) + (d(a_hi, b_lo) + d(a_lo, b_hi))


def _dft_block_rect(row0):
    r_idx = lax.broadcasted_iota(jnp.int32, (FB, NF), 0) + row0
    f_idx = lax.broadcasted_iota(jnp.int32, (FB, NF), 1)
    prod = (r_idx * f_idx) & (S - 1)
    theta = prod.astype(jnp.float32) * _TWO_PI_OVER_S
    return jnp.cos(theta), jnp.sin(theta)


def _fwd1_kernel(qa_ref, ka_ref, sr_ref, si_ref, nyq_ref):
    fb = pl.program_id(0)
    fc, fs = _dft_block(fb * FB)
    qa = jnp.concatenate([qa_ref[b] for b in range(B)], axis=1)
    ka = jnp.concatenate([ka_ref[b] for b in range(B)], axis=1)
    qr = _dot3(fc, qa)
    qi = -_dot3(fs, qa)
    kr = _dot3(fc, ka)
    ki = -_dot3(fs, ka)
    f_col = lax.broadcasted_iota(jnp.int32, (FB, 1), 0) + fb * FB
    sc = jnp.where(f_col == 0, 1.0 / S, 2.0 / S)
    sr_ref[...] = (qr * kr + qi * ki) * sc
    si_ref[...] = (qi * kr - qr * ki) * sc

    @pl.when(fb == 0)
    def _():
        alt = (1 - 2 * (lax.broadcasted_iota(jnp.int32, (S, 1), 0) & 1)
               ).astype(jnp.float32)
        qn = jnp.sum(qa * alt, axis=0, keepdims=True)
        kn = jnp.sum(ka * alt, axis=0, keepdims=True)
        nyq_ref[...] = qn * kn * (1.0 / S)


def _inv1_kernel(sr_ref, si_ref, nyq_ref, abs_ref):
    tb = pl.program_id(0)
    fc, fs = _dft_block_rect(tb * FB)
    c = _dot3(fc, sr_ref[...])
    d = _dot3(fs, si_ref[...])
    alt = (1 - 2 * (lax.broadcasted_iota(jnp.int32, (FB, 1), 0) + tb * FB & 1)
           ).astype(jnp.float32)
    abs_ref[...] = jnp.abs(c - d + alt * nyq_ref[...])


def _topk_kernel(abs_ref, w_ref, lag_ref):
    x = abs_ref[...]
    iota = lax.broadcasted_iota(jnp.int32, (S, NR), 0)
    vals = []
    lags = []
    for _ in range(KK):
        m = jnp.max(x, axis=0, keepdims=True)
        hit = x == m
        am = jnp.min(jnp.where(hit, iota, S), axis=0, keepdims=True)
        vals.append(m)
        lags.append(am)
        x = jnp.where(iota == am, -1.0, x)
    v15 = jnp.concatenate(vals, axis=0)
    e = jnp.exp(v15 - v15[0:1])
    w15 = e / jnp.sum(e, axis=0, keepdims=True)
    w_ref[...] = jnp.concatenate(
        [w15, jnp.zeros((1, NR), jnp.float32)], axis=0)
    lag_ref[...] = jnp.concatenate(
        lags + [jnp.zeros((1, NR), jnp.int32)], axis=0)


@functools.cache
def _make_agg_sc():
    mesh = plsc.VectorSubcoreMesh(core_axis_name="c", subcore_axis_name="s",
                                  num_cores=SC_NC)

    @functools.partial(
        pl.kernel,
        out_type=jax.ShapeDtypeStruct((NR, S), jnp.float32),
        mesh=mesh,
        compiler_params=pltpu.CompilerParams(needs_layout_passes=False),
        scratch_types=[
            pltpu.VMEM((S,), jnp.float32),
            pltpu.VMEM((S,), jnp.float32),
            pltpu.VMEM((16 * 16,), jnp.int32),
            pltpu.VMEM((16 * 16,), jnp.float32),
        ],
    )
    def agg(vt_hbm, lag_hbm, w_hbm, out_hbm, vrow, orow, lrow, wrow):
        wid = lax.axis_index("s") * SC_NC + lax.axis_index("c")
        base = wid * ROWS_PER_W
        lane = lax.iota(jnp.int32, 16)

        def row_body(j, carry):
            r = base + j
            pltpu.sync_copy(vt_hbm.at[r], vrow)
            pltpu.sync_copy(lag_hbm.at[r], lrow)
            pltpu.sync_copy(w_hbm.at[r], wrow)
            lag_b = [lrow[pl.ds(16 * i, 16)] for i in range(16)]
            w_b = [wrow[pl.ds(16 * i, 16)] for i in range(16)]

            def chunk_body(jc, carry2):
                basei = lane + jc * 16
                acc = jnp.zeros((16,), jnp.float32)
                for i in range(16):
                    idx = (basei + lag_b[i]) & (S - 1)
                    acc = acc + plsc.load_gather(vrow, [idx]) * w_b[i]
                orow[pl.ds(jc * 16, 16)] = acc
                return carry2

            lax.fori_loop(0, S // 16, chunk_body, 0)
            pltpu.sync_copy(orow, out_hbm.at[r])
            return carry

        lax.fori_loop(0, ROWS_PER_W, row_body, 0)

    return agg


def _tile_kernel(agg_ref, out_ref):
    a = agg_ref[0]
    eye = jnp.eye(DK, dtype=jnp.float32)
    at = lax.dot_general(a, eye, (((0,), (0,)), ((), ())),
                         preferred_element_type=jnp.float32,
                         precision=PREC)
    out_ref[0] = jnp.concatenate([at] * H, axis=1)


def kernel(q_in, k_in, v_in, Wq, bq):
    dch = D // DCH
    qa, ka, vt = pl.pallas_call(
        _proj_kernel,
        grid=(B, DCH),
        in_specs=[
            pl.BlockSpec((1, S, dch), lambda b, d: (b, 0, d)),
            pl.BlockSpec((1, S, dch), lambda b, d: (b, 0, d)),
            pl.BlockSpec((1, S, dch), lambda b, d: (b, 0, d)),
            pl.BlockSpec((dch, DK), lambda b, d: (d, 0)),
            pl.BlockSpec((1, DK), lambda b, d: (0, 0)),
            pl.BlockSpec((DK, 1), lambda b, d: (0, 0)),
        ],
        out_specs=[
            pl.BlockSpec((1, S, DK), lambda b, d: (b, 0, 0)),
            pl.BlockSpec((1, S, DK), lambda b, d: (b, 0, 0)),
            pl.BlockSpec((1, DK, S), lambda b, d: (b, 0, 0)),
        ],
        out_shape=[
            jax.ShapeDtypeStruct((B, S, DK), jnp.float32),
            jax.ShapeDtypeStruct((B, S, DK), jnp.float32),
            jax.ShapeDtypeStruct((B, DK, S), jnp.float32),
        ],
    )(q_in, k_in, v_in, Wq, bq.reshape(1, DK), bq.reshape(DK, 1))

    sr, si, nyq = pl.pallas_call(
        _fwd1_kernel,
        grid=(NFB2,),
        in_specs=[
            pl.BlockSpec((B, S, DK), lambda f: (0, 0, 0)),
            pl.BlockSpec((B, S, DK), lambda f: (0, 0, 0)),
        ],
        out_specs=[
            pl.BlockSpec((FB, NR), lambda f: (f, 0)),
            pl.BlockSpec((FB, NR), lambda f: (f, 0)),
            pl.BlockSpec((1, NR), lambda f: (0, 0)),
        ],
        out_shape=[
            jax.ShapeDtypeStruct((NF, NR), jnp.float32),
            jax.ShapeDtypeStruct((NF, NR), jnp.float32),
            jax.ShapeDtypeStruct((1, NR), jnp.float32),
        ],
    )(qa, ka)

    qk_abs = pl.pallas_call(
        _inv1_kernel,
        grid=(NFB,),
        in_specs=[
            pl.BlockSpec((NF, NR), lambda t: (0, 0)),
            pl.BlockSpec((NF, NR), lambda t: (0, 0)),
            pl.BlockSpec((1, NR), lambda t: (0, 0)),
        ],
        out_specs=pl.BlockSpec((FB, NR), lambda t: (t, 0)),
        out_shape=jax.ShapeDtypeStruct((S, NR), jnp.float32),
    )(sr, si, nyq)

    w16, lag16 = pl.pallas_call(
        _topk_kernel,
        out_shape=[
            jax.ShapeDtypeStruct((16, NR), jnp.float32),
            jax.ShapeDtypeStruct((16, NR), jnp.int32),
        ],
    )(qk_abs)

    lag_bc = jnp.broadcast_to(lag16.T[:, :, None], (NR, 16, 16)).reshape(NR, 256)
    w_bc = jnp.broadcast_to(w16.T[:, :, None], (NR, 16, 16)).reshape(NR, 256)
    agg = _make_agg_sc()(vt.reshape(NR, S), lag_bc, w_bc)

    out = pl.pallas_call(
        _tile_kernel,
        grid=(B,),
        in_specs=[pl.BlockSpec((1, DK, S), lambda b: (b, 0, 0))],
        out_specs=pl.BlockSpec((1, S, H * DK), lambda b: (b, 0, 0)),
        out_shape=jax.ShapeDtypeStruct((B, S, H * DK), jnp.float32),
    )(agg.reshape(B, DK, S))
    return out


# speedup vs baseline: 61.5258x; 1.0160x over previous
import functools

import jax
import jax.numpy as jnp
import numpy as np
from jax import lax
from jax.experimental import pallas as pl
from jax.experimental.pallas import tpu as pltpu
from jax.experimental.pallas import tpu_sc as plsc

B, S, D = 4, 2048, 1024
DK = 64
H = 16
KK = 15
NR = B * DK
DCH = 4
FB = 256
NFB = S // FB
PREC = lax.Precision.HIGHEST
NF = S // 2
NFB2 = NF // FB
_TWO_PI_OVER_S = float(2.0 * np.pi / S)

SC_NC = 2
SC_NS = 16
SC_NW = SC_NC * SC_NS
ROWS_PER_W = NR // SC_NW


def _proj_kernel(q_ref, k_ref, v_ref, wq_ref, bqr_ref, bqc_ref,
                 qa_ref, ka_ref, vt_ref):
    d = pl.program_id(1)
    w = wq_ref[...].astype(jnp.bfloat16)
    qp = jnp.dot(q_ref[0].astype(jnp.bfloat16), w,
                 preferred_element_type=jnp.float32)
    kp = jnp.dot(k_ref[0].astype(jnp.bfloat16), w,
                 preferred_element_type=jnp.float32)
    vp = lax.dot_general(w, v_ref[0].astype(jnp.bfloat16),
                         (((0,), (1,)), ((), ())),
                         preferred_element_type=jnp.float32)

    @pl.when(d == 0)
    def _():
        qa_ref[0] = qp + bqr_ref[...]
        ka_ref[0] = kp + bqr_ref[...]
        vt_ref[0] = vp + bqc_ref[...]

    @pl.when(d != 0)
    def _():
        qa_ref[0] = qa_ref[0] + qp
        ka_ref[0] = ka_ref[0] + kp
        vt_ref[0] = vt_ref[0] + vp


def _dft_block(row0):
    r_idx = lax.broadcasted_iota(jnp.int32, (FB, S), 0) + row0
    t_idx = lax.broadcasted_iota(jnp.int32, (FB, S), 1)
    prod = (r_idx * t_idx) & (S - 1)
    theta = prod.astype(jnp.float32) * _TWO_PI_OVER_S
    return jnp.cos(theta), jnp.sin(theta)


def _dot3(a, b):
    a_hi = a.astype(jnp.bfloat16)
    a_lo = (a - a_hi.astype(jnp.float32)).astype(jnp.bfloat16)
    b_hi = b.astype(jnp.bfloat16)
    b_lo = (b - b_hi.astype(jnp.float32)).astype(jnp.bfloat16)

    def d(x, y):
        return jnp.dot(x, y, preferred_element_type=jnp.float32)

    return d(a_hi, b_hi) + (d(a_hi, b_lo) + d(a_lo, b_hi))


def _dft_block_rect(row0):
    r_idx = lax.broadcasted_iota(jnp.int32, (FB, NF), 0) + row0
    f_idx = lax.broadcasted_iota(jnp.int32, (FB, NF), 1)
    prod = (r_idx * f_idx) & (S - 1)
    theta = prod.astype(jnp.float32) * _TWO_PI_OVER_S
    return jnp.cos(theta), jnp.sin(theta)


def _fwd1_kernel(qa_ref, ka_ref, sr_ref, si_ref, nyq_ref):
    fb = pl.program_id(0)
    fc, fs = _dft_block(fb * FB)
    qa = jnp.concatenate([qa_ref[b] for b in range(B)], axis=1)
    ka = jnp.concatenate([ka_ref[b] for b in range(B)], axis=1)
    qr = _dot3(fc, qa)
    qi = -_dot3(fs, qa)
    kr = _dot3(fc, ka)
    ki = -_dot3(fs, ka)
    f_col = lax.broadcasted_iota(jnp.int32, (FB, 1), 0) + fb * FB
    sc = jnp.where(f_col == 0, 1.0 / S, 2.0 / S)
    sr_ref[...] = (qr * kr + qi * ki) * sc
    si_ref[...] = (qi * kr - qr * ki) * sc

    @pl.when(fb == 0)
    def _():
        alt = (1 - 2 * (lax.broadcasted_iota(jnp.int32, (S, 1), 0) & 1)
               ).astype(jnp.float32)
        qn = jnp.sum(qa * alt, axis=0, keepdims=True)
        kn = jnp.sum(ka * alt, axis=0, keepdims=True)
        nyq_ref[...] = qn * kn * (1.0 / S)


def _inv1_kernel(sr_ref, si_ref, nyq_ref, abs_ref):
    tb = pl.program_id(0)
    fc, fs = _dft_block_rect(tb * FB)
    c = _dot3(fc, sr_ref[...])
    d = _dot3(fs, si_ref[...])
    alt = (1 - 2 * (lax.broadcasted_iota(jnp.int32, (FB, 1), 0) + tb * FB & 1)
           ).astype(jnp.float32)
    abs_ref[...] = jnp.abs(c - d + alt * nyq_ref[...])


def _topk_kernel(abs_ref, w_ref, lag_ref):
    x = abs_ref[...]
    iota = lax.broadcasted_iota(jnp.int32, (S, NR), 0)
    vals = []
    lags = []
    for _ in range(KK):
        m = jnp.max(x, axis=0, keepdims=True)
        hit = x == m
        am = jnp.min(jnp.where(hit, iota, S), axis=0, keepdims=True)
        vals.append(m)
        lags.append(am)
        x = jnp.where(iota == am, -1.0, x)
    v15 = jnp.concatenate(vals, axis=0)
    e = jnp.exp(v15 - v15[0:1])
    w15 = e / jnp.sum(e, axis=0, keepdims=True)
    w16 = jnp.concatenate(
        [w15, jnp.zeros((1, NR), jnp.float32)], axis=0)
    lag16 = jnp.concatenate(
        lags + [jnp.zeros((1, NR), jnp.int32)], axis=0)
    sel = (lax.broadcasted_iota(jnp.int32, (16, 256), 1) // 16
           == lax.broadcasted_iota(jnp.int32, (16, 256), 0)
           ).astype(jnp.float32)
    w_ref[...] = lax.dot_general(w16, sel, (((0,), (0,)), ((), ())),
                                 preferred_element_type=jnp.float32,
                                 precision=PREC)
    lagf = lax.dot_general(lag16.astype(jnp.float32), sel,
                           (((0,), (0,)), ((), ())),
                           preferred_element_type=jnp.float32,
                           precision=PREC)
    lag_ref[...] = (lagf + 0.5).astype(jnp.int32)


@functools.cache
def _make_agg_sc():
    mesh = plsc.VectorSubcoreMesh(core_axis_name="c", subcore_axis_name="s",
                                  num_cores=SC_NC)

    @functools.partial(
        pl.kernel,
        out_type=jax.ShapeDtypeStruct((NR, S), jnp.float32),
        mesh=mesh,
        compiler_params=pltpu.CompilerParams(needs_layout_passes=False),
        scratch_types=[
            pltpu.VMEM((S,), jnp.float32),
            pltpu.VMEM((S,), jnp.float32),
            pltpu.VMEM((16 * 16,), jnp.int32),
            pltpu.VMEM((16 * 16,), jnp.float32),
        ],
    )
    def agg(vt_hbm, lag_hbm, w_hbm, out_hbm, vrow, orow, lrow, wrow):
        wid = lax.axis_index("s") * SC_NC + lax.axis_index("c")
        base = wid * ROWS_PER_W
        lane = lax.iota(jnp.int32, 16)

        def row_body(j, carry):
            r = base + j
            pltpu.sync_copy(vt_hbm.at[r], vrow)
            pltpu.sync_copy(lag_hbm.at[r], lrow)
            pltpu.sync_copy(w_hbm.at[r], wrow)
            lag_b = [lrow[pl.ds(16 * i, 16)] for i in range(16)]
            w_b = [wrow[pl.ds(16 * i, 16)] for i in range(16)]

            def chunk_body(jc, carry2):
                basei = lane + jc * 16
                acc = jnp.zeros((16,), jnp.float32)
                for i in range(16):
                    idx = (basei + lag_b[i]) & (S - 1)
                    acc = acc + plsc.load_gather(vrow, [idx]) * w_b[i]
                orow[pl.ds(jc * 16, 16)] = acc
                return carry2

            lax.fori_loop(0, S // 16, chunk_body, 0)
            pltpu.sync_copy(orow, out_hbm.at[r])
            return carry

        lax.fori_loop(0, ROWS_PER_W, row_body, 0)

    return agg


def _tile_kernel(agg_ref, out_ref):
    a = agg_ref[0]
    eye = jnp.eye(DK, dtype=jnp.float32)
    at = lax.dot_general(a, eye, (((0,), (0,)), ((), ())),
                         preferred_element_type=jnp.float32,
                         precision=PREC)
    out_ref[0] = jnp.concatenate([at] * H, axis=1)


def kernel(q_in, k_in, v_in, Wq, bq):
    dch = D // DCH
    qa, ka, vt = pl.pallas_call(
        _proj_kernel,
        grid=(B, DCH),
        in_specs=[
            pl.BlockSpec((1, S, dch), lambda b, d: (b, 0, d)),
            pl.BlockSpec((1, S, dch), lambda b, d: (b, 0, d)),
            pl.BlockSpec((1, S, dch), lambda b, d: (b, 0, d)),
            pl.BlockSpec((dch, DK), lambda b, d: (d, 0)),
            pl.BlockSpec((1, DK), lambda b, d: (0, 0)),
            pl.BlockSpec((DK, 1), lambda b, d: (0, 0)),
        ],
        out_specs=[
            pl.BlockSpec((1, S, DK), lambda b, d: (b, 0, 0)),
            pl.BlockSpec((1, S, DK), lambda b, d: (b, 0, 0)),
            pl.BlockSpec((1, DK, S), lambda b, d: (b, 0, 0)),
        ],
        out_shape=[
            jax.ShapeDtypeStruct((B, S, DK), jnp.float32),
            jax.ShapeDtypeStruct((B, S, DK), jnp.float32),
            jax.ShapeDtypeStruct((B, DK, S), jnp.float32),
        ],
    )(q_in, k_in, v_in, Wq, bq.reshape(1, DK), bq.reshape(DK, 1))

    sr, si, nyq = pl.pallas_call(
        _fwd1_kernel,
        grid=(NFB2,),
        in_specs=[
            pl.BlockSpec((B, S, DK), lambda f: (0, 0, 0)),
            pl.BlockSpec((B, S, DK), lambda f: (0, 0, 0)),
        ],
        out_specs=[
            pl.BlockSpec((FB, NR), lambda f: (f, 0)),
            pl.BlockSpec((FB, NR), lambda f: (f, 0)),
            pl.BlockSpec((1, NR), lambda f: (0, 0)),
        ],
        out_shape=[
            jax.ShapeDtypeStruct((NF, NR), jnp.float32),
            jax.ShapeDtypeStruct((NF, NR), jnp.float32),
            jax.ShapeDtypeStruct((1, NR), jnp.float32),
        ],
    )(qa, ka)

    qk_abs = pl.pallas_call(
        _inv1_kernel,
        grid=(NFB,),
        in_specs=[
            pl.BlockSpec((NF, NR), lambda t: (0, 0)),
            pl.BlockSpec((NF, NR), lambda t: (0, 0)),
            pl.BlockSpec((1, NR), lambda t: (0, 0)),
        ],
        out_specs=pl.BlockSpec((FB, NR), lambda t: (t, 0)),
        out_shape=jax.ShapeDtypeStruct((S, NR), jnp.float32),
    )(sr, si, nyq)

    w_bc, lag_bc = pl.pallas_call(
        _topk_kernel,
        out_shape=[
            jax.ShapeDtypeStruct((NR, 256), jnp.float32),
            jax.ShapeDtypeStruct((NR, 256), jnp.int32),
        ],
    )(qk_abs)

    agg = _make_agg_sc()(vt.reshape(NR, S), lag_bc, w_bc)

    out = pl.pallas_call(
        _tile_kernel,
        grid=(B,),
        in_specs=[pl.BlockSpec((1, DK, S), lambda b: (b, 0, 0))],
        out_specs=pl.BlockSpec((1, S, H * DK), lambda b: (b, 0, 0)),
        out_shape=jax.ShapeDtypeStruct((B, S, H * DK), jnp.float32),
    )(agg.reshape(B, DK, S))
    return out
